# Initial kernel scaffold; baseline (speedup 1.0000x reference)
import jax
import jax.numpy as jnp
from jax.experimental import pallas as pl


def kernel(mem, idx, val):
    raise NotImplementedError("write your pallas kernel here")


# speedup vs baseline: 33.8699x; 33.8699x over previous
import functools

import jax
import jax.numpy as jnp
from jax import lax
from jax.experimental import pallas as pl
from jax.experimental.pallas import tpu as pltpu
from jax.experimental.pallas import tpu_sc as plsc

_L = 16
_NC = 2
_NS = 16
_NW = _NC * _NS
_SHARD = 65536
_RW = 128
_TROWS = _NS * _SHARD // _RW


def _shift_up_one(x, lane):
    perm = jnp.minimum(lane + 1, _L - 1)
    dn = lax.GatherDimensionNumbers(
        offset_dims=(), collapsed_slice_dims=(0,), start_index_map=(0,))
    return lax.gather(x, perm[:, None], dn, slice_sizes=(1,),
                      mode=lax.GatherScatterMode.PROMISE_IN_BOUNDS)


def _make_sc_kernel(M, B, D):
    n_vregs = B // _L
    nb = B // _NW
    gb = 128
    mesh = plsc.VectorSubcoreMesh(core_axis_name="c", subcore_axis_name="s")

    @functools.partial(
        pl.kernel,
        mesh=mesh,
        compiler_params=pltpu.CompilerParams(needs_layout_passes=False),
        out_type=(
            jax.ShapeDtypeStruct((B, D), jnp.float32),
            jax.ShapeDtypeStruct((_NC, _TROWS, _RW), jnp.int32),
        ),
        scratch_types=[
            pltpu.VMEM((B,), jnp.int32),
            pltpu.VMEM((_SHARD // _RW, _RW), jnp.int32),
            pltpu.VMEM((nb,), jnp.int32),
            pltpu.VMEM((gb, _RW), jnp.int32),
            pltpu.VMEM((nb,), jnp.int32),
            pltpu.VMEM((gb, D), jnp.float32),
            pltpu.SemaphoreType.DMA,
        ],
    )
    def sc_kernel(idx_hbm, val_hbm, out_hbm, table_hbm, idx_v, tslice,
                  rowid_v, trows_v, w_v, rows_v, sem):
        c = lax.axis_index("c")
        s = lax.axis_index("s")
        wid = s * _NC + c

        pltpu.sync_copy(idx_hbm, idx_v)
        lane = lax.iota(jnp.int32, _L)

        def body(k, carry):
            v = idx_v[pl.ds(k * _L, _L)]
            key = v * _L + lane
            pos = k * _L + lane
            skey, spos = plsc.sort_key_val(key, pos)
            sidx = skey >> 4
            nxt = _shift_up_one(sidx, lane)
            is_win = (lane == (_L - 1)) | (sidx != nxt)
            mine = (sidx >> 16) == s
            rel = sidx & (_SHARD - 1)
            plsc.store_scatter(tslice, [rel >> 7, rel & (_RW - 1)], spos,
                               mask=is_win & mine)
            return carry

        lax.fori_loop(0, n_vregs, body, None)

        rows_per_shard = _SHARD // _RW
        pltpu.sync_copy(
            tslice, table_hbm.at[c, pl.ds(s * rows_per_shard,
                                          rows_per_shard)])
        plsc.subcore_barrier()

        base = wid * nb

        def rowids(k, carry):
            v = idx_v[pl.ds(base + k * _L, _L)]
            rowid_v[pl.ds(k * _L, _L)] = v >> 7
            return carry

        lax.fori_loop(0, nb // _L, rowids, None)

        for g in range(nb // gb):
            pltpu.async_copy(
                table_hbm.at[c].at[rowid_v.at[pl.ds(g * gb, gb)]], trows_v,
                sem).wait()
            for k in range(gb // _L):
                e = g * gb + k * _L
                v = idx_v[pl.ds(base + e, _L)]
                w = plsc.load_gather(trows_v,
                                     [k * _L + lane, v & (_RW - 1)])
                w_v[pl.ds(e, _L)] = w

        for t in range(nb // gb):
            pltpu.async_copy(
                val_hbm.at[w_v.at[pl.ds(t * gb, gb)]], rows_v, sem).wait()
            pltpu.sync_copy(rows_v, out_hbm.at[pl.ds(base + t * gb, gb)])

    return sc_kernel


def kernel(mem, idx, val):
    del mem
    M = 1000000
    B, D = val.shape
    out, _table = _make_sc_kernel(M, B, D)(idx, val)
    return out


# speedup vs baseline: 36.4587x; 1.0764x over previous
import functools

import jax
import jax.numpy as jnp
from jax import lax
from jax.experimental import pallas as pl
from jax.experimental.pallas import tpu as pltpu
from jax.experimental.pallas import tpu_sc as plsc

_L = 16
_NC = 2
_NS = 16
_NW = _NC * _NS
_SHARD = 65536
_RW = 128
_TROWS = _NS * _SHARD // _RW


def _shift_up_one(x, lane):
    perm = jnp.minimum(lane + 1, _L - 1)
    dn = lax.GatherDimensionNumbers(
        offset_dims=(), collapsed_slice_dims=(0,), start_index_map=(0,))
    return lax.gather(x, perm[:, None], dn, slice_sizes=(1,),
                      mode=lax.GatherScatterMode.PROMISE_IN_BOUNDS)


def _make_sc_kernel(M, B, D):
    n_vregs = B // _L
    unroll = 4
    nb = B // _NW
    gb = 64
    ng = nb // gb
    mesh = plsc.VectorSubcoreMesh(core_axis_name="c", subcore_axis_name="s")

    @functools.partial(
        pl.kernel,
        mesh=mesh,
        compiler_params=pltpu.CompilerParams(needs_layout_passes=False),
        out_type=(
            jax.ShapeDtypeStruct((B, D), jnp.float32),
            jax.ShapeDtypeStruct((_NC, _TROWS, _RW), jnp.int32),
        ),
        scratch_types=[
            pltpu.VMEM((B,), jnp.int32),
            pltpu.VMEM((_SHARD // _RW, _RW), jnp.int32),
            pltpu.VMEM((nb,), jnp.int32),
            pltpu.VMEM((2, gb, _RW), jnp.int32),
            pltpu.VMEM((nb,), jnp.int32),
            pltpu.VMEM((2, gb, D), jnp.float32),
            pltpu.SemaphoreType.DMA,
            pltpu.SemaphoreType.DMA,
            pltpu.SemaphoreType.DMA,
            pltpu.SemaphoreType.DMA,
        ],
    )
    def sc_kernel(idx_hbm, val_hbm, out_hbm, table_hbm, idx_v, tslice,
                  rowid_v, trows_v, w_v, rows_v, tsem0, tsem1, vsem0, vsem1):
        c = lax.axis_index("c")
        s = lax.axis_index("s")
        wid = s * _NC + c
        tsems = (tsem0, tsem1)
        vsems = (vsem0, vsem1)

        pltpu.sync_copy(idx_hbm, idx_v)
        lane = lax.iota(jnp.int32, _L)

        def scan_one(k):
            v = idx_v[pl.ds(k * _L, _L)]
            key = v * _L + lane
            pos = k * _L + lane
            skey, spos = plsc.sort_key_val(key, pos)
            sidx = skey >> 4
            nxt = _shift_up_one(sidx, lane)
            is_win = (lane == (_L - 1)) | (sidx != nxt)
            mine = (sidx >> 16) == s
            rel = sidx & (_SHARD - 1)
            plsc.store_scatter(tslice, [rel >> 7, rel & (_RW - 1)], spos,
                               mask=is_win & mine)

        def body(k, carry):
            for u in range(unroll):
                scan_one(k * unroll + u)
            return carry

        lax.fori_loop(0, n_vregs // unroll, body, None)

        rows_per_shard = _SHARD // _RW
        pltpu.sync_copy(
            tslice, table_hbm.at[c, pl.ds(s * rows_per_shard,
                                          rows_per_shard)])

        base = wid * nb

        def rowids(k, carry):
            v = idx_v[pl.ds(base + k * _L, _L)]
            rowid_v[pl.ds(k * _L, _L)] = v >> 7
            return carry

        lax.fori_loop(0, nb // _L, rowids, None)
        plsc.subcore_barrier()

        def start_tgather(g):
            return pltpu.async_copy(
                table_hbm.at[c].at[rowid_v.at[pl.ds(g * gb, gb)]],
                trows_v.at[g % 2], tsems[g % 2])

        def start_vgather(g):
            return pltpu.async_copy(
                val_hbm.at[w_v.at[pl.ds(g * gb, gb)]],
                rows_v.at[g % 2], vsems[g % 2])

        tcopies = {0: start_tgather(0), 1: start_tgather(1)}
        vcopies = {}
        for g in range(ng):
            tcopies.pop(g).wait()
            for k in range(gb // _L):
                e = g * gb + k * _L
                v = idx_v[pl.ds(base + e, _L)]
                w = plsc.load_gather(trows_v.at[g % 2],
                                     [k * _L + lane, v & (_RW - 1)])
                w_v[pl.ds(e, _L)] = w
            if g + 2 < ng:
                tcopies[g + 2] = start_tgather(g + 2)
            vcopies[g] = start_vgather(g)
            if g > 0:
                vcopies.pop(g - 1).wait()
                pltpu.sync_copy(rows_v.at[(g - 1) % 2],
                                out_hbm.at[pl.ds(base + (g - 1) * gb, gb)])
        vcopies.pop(ng - 1).wait()
        pltpu.sync_copy(rows_v.at[(ng - 1) % 2],
                        out_hbm.at[pl.ds(base + (ng - 1) * gb, gb)])

    return sc_kernel


def kernel(mem, idx, val):
    del mem
    M = 1000000
    B, D = val.shape
    out, _table = _make_sc_kernel(M, B, D)(idx, val)
    return out
